# baseline (speedup 1.0000x reference)
import functools

import jax
import jax.numpy as jnp
from jax import lax
from jax.experimental import pallas as pl
from jax.experimental.pallas import tpu as pltpu
from jax.experimental.pallas import tpu_sc as plsc

N = 10000
E = 320000
IN_CH = 128
HID_CH = 256
OUT_CH = 128

NC = 2
NS = 16
L = 16

K = 128
SK = K // 2
CHT = 160
EPT = CHT * K
EPAD = NS * EPT
G = 8
GK = G * K
NGRP = CHT // G
NBUF = 2
RPT_A = 624
RPT_LAST = N - (NS - 1) * RPT_A


def _mm1_body(x_ref, w_ref, o0_ref, o1_ref):
    x = x_ref[...]
    o0_ref[...] = jnp.dot(
        x, w_ref[0], preferred_element_type=jnp.float32).astype(jnp.bfloat16)
    o1_ref[...] = jnp.dot(
        x, w_ref[1], preferred_element_type=jnp.float32).astype(jnp.bfloat16)


def _mm2_body(a_ref, b1_ref, w_ref, o0_ref, o1_ref):
    a0 = jnp.maximum(a_ref[0] + b1_ref[0], 0.0)
    a1 = jnp.maximum(a_ref[1] + b1_ref[1], 0.0)
    h = (jnp.dot(a0, w_ref[0], preferred_element_type=jnp.float32)
         + jnp.dot(a1, w_ref[1], preferred_element_type=jnp.float32))
    o0_ref[...] = h[:, :OUT_CH // 2].astype(jnp.bfloat16)
    o1_ref[...] = h[:, OUT_CH // 2:].astype(jnp.bfloat16)


def _make_agg(C):
    mesh = plsc.VectorSubcoreMesh(core_axis_name="c", subcore_axis_name="s")

    @functools.partial(
        pl.kernel,
        out_type=jax.ShapeDtypeStruct((NC, N, C), jnp.float32),
        mesh=mesh,
        compiler_params=pltpu.CompilerParams(
            needs_layout_passes=False, use_tc_tiling_on_sc=False),
        scratch_types=[
            pltpu.VMEM((2, G, K), jnp.int32),
            pltpu.VMEM((2, 2 * G, SK), jnp.int32),
            pltpu.VMEM((2 * GK,), jnp.float32),
            pltpu.VMEM((NBUF, K, C), jnp.bfloat16),
            pltpu.VMEM((2, SK, C), jnp.float32),
            pltpu.VMEM_SHARED((N, C), jnp.float32),
            [pltpu.SemaphoreType.DMA] * NBUF,
            [pltpu.SemaphoreType.DMA] * 2,
            [pltpu.SemaphoreType.DMA] * 2,
            [pltpu.SemaphoreType.DMA] * 2,
            [pltpu.SemaphoreType.DMA] * 2,
        ],
    )
    def agg(tab0, tab1, src2d, dst2d, wflat, init, out,
            srcv, dstv, wv, rows, outb, acc, gsems, ssems, isems, jsems,
            ksems):
        cid = lax.axis_index("c")
        sid = lax.axis_index("s")
        stripe_a = pl.ds(sid * RPT_A, RPT_A)
        stripe_l = pl.ds((NS - 1) * RPT_A, RPT_LAST)

        @pl.when(sid < NS - 1)
        def _():
            pltpu.sync_copy(init.at[cid, pl.ds(0, RPT_A)], acc.at[stripe_a])

        @pl.when(sid == NS - 1)
        def _():
            pltpu.sync_copy(init.at[cid], acc.at[stripe_l])

        def istart(g, p):
            r0 = sid * CHT + g * G
            pltpu.async_copy(src2d.at[pl.ds(r0, G)], srcv.at[p], isems[p])
            pltpu.async_copy(
                dst2d.at[pl.ds(2 * r0, 2 * G)], dstv.at[p], jsems[p])
            pltpu.async_copy(wflat.at[pl.ds(r0 * K, GK)],
                             wv.at[pl.ds(p * GK, GK)], ksems[p])

        def iwait(p):
            pltpu.make_async_copy(
                src2d.at[pl.ds(0, G)], srcv.at[p], isems[p]).wait()
            pltpu.make_async_copy(
                dst2d.at[pl.ds(0, 2 * G)], dstv.at[p], jsems[p]).wait()
            pltpu.make_async_copy(
                wflat.at[pl.ds(0, GK)], wv.at[pl.ds(p * GK, GK)],
                ksems[p]).wait()

        def gstart(p, brow, q):
            @pl.when(cid == 0)
            def _():
                pltpu.async_copy(
                    tab0.at[srcv.at[p, brow]], rows.at[q], gsems[q])

            @pl.when(cid == 1)
            def _():
                pltpu.async_copy(
                    tab1.at[srcv.at[p, brow]], rows.at[q], gsems[q])

        def gwait(q):
            pltpu.make_async_copy(
                tab0.at[srcv.at[0, 0]], rows.at[q], gsems[q]).wait()

        def sstart(p, hrow, o):
            pltpu.async_copy(
                outb.at[o], acc.at[dstv.at[p, hrow]], ssems[o], add=True)

        def swait_out(o):
            pltpu.make_async_copy(
                outb.at[o], acc.at[dstv.at[0, 0]], ssems[o]).wait()

        def scale_half(p, b, q, sub):
            wbase = p * GK + b * K

            @pl.loop(sub * SK, (sub + 1) * SK, unroll=4)
            def _edge(i):
                wspl = plsc.load_gather(
                    wv, [jnp.full((L,), wbase + i, jnp.int32)])
                io = i - sub * SK
                for cb in range(C // (2 * L)):
                    v = rows[q, i, pl.ds(cb * 2 * L, 2 * L)]
                    va, vb = plsc.unpack(
                        v, format=plsc.PackFormat.INTERLEAVED)
                    outb[sub, io, pl.ds(cb * 2 * L, L)] = va * wspl
                    outb[sub, io, pl.ds(cb * 2 * L + L, L)] = vb * wspl

        istart(0, 0)
        istart(1, 1)
        iwait(0)
        gstart(0, 0, 0)
        plsc.subcore_barrier()

        @pl.loop(0, NGRP, step=2)
        def _grp(g):
            for pp in range(2):
                gg = g + pp
                for b in range(G):
                    t = gg * G + b
                    q = b % NBUF
                    pn = pp if b < G - 1 else 1 - pp
                    brow = (b + 1) % G

                    @pl.when(t + 1 < CHT)
                    def _():
                        gstart(pn, brow, (b + 1) % NBUF)

                    gwait(q)
                    for sub in range(2):
                        @pl.when(t >= 1)
                        def _():
                            swait_out(sub)

                        scale_half(pp, b, q, sub)
                        sstart(pp, 2 * b + sub, sub)
                    if b == 2:
                        @pl.when((gg >= 1) & (gg + 1 < NGRP))
                        def _():
                            istart(gg + 1, 1 - pp)
                    if b == 6:
                        @pl.when(gg + 1 < NGRP)
                        def _():
                            iwait(1 - pp)

        swait_out(0)
        swait_out(1)
        plsc.subcore_barrier()

        @pl.when(sid < NS - 1)
        def _():
            pltpu.sync_copy(acc.at[stripe_a], out.at[cid, stripe_a])

        @pl.when(sid == NS - 1)
        def _():
            pltpu.sync_copy(acc.at[stripe_l], out.at[cid, stripe_l])

    return agg


_agg_hid = _make_agg(HID_CH // 2)
_agg_out = _make_agg(OUT_CH // 2)

_BN = 1000

_mm1 = pl.pallas_call(
    _mm1_body,
    grid=(N // _BN,),
    in_specs=[
        pl.BlockSpec((_BN, IN_CH), lambda i: (i, 0)),
        pl.BlockSpec((2, IN_CH, HID_CH // 2), lambda i: (0, 0, 0)),
    ],
    out_specs=[
        pl.BlockSpec((_BN, HID_CH // 2), lambda i: (i, 0)),
        pl.BlockSpec((_BN, HID_CH // 2), lambda i: (i, 0)),
    ],
    out_shape=[jax.ShapeDtypeStruct((N, HID_CH // 2), jnp.bfloat16)] * 2,
)

_mm2 = pl.pallas_call(
    _mm2_body,
    grid=(N // _BN,),
    in_specs=[
        pl.BlockSpec((2, _BN, HID_CH // 2), lambda i: (0, i, 0)),
        pl.BlockSpec((2, 1, HID_CH // 2), lambda i: (0, 0, 0)),
        pl.BlockSpec((2, HID_CH // 2, OUT_CH), lambda i: (0, 0, 0)),
    ],
    out_specs=[
        pl.BlockSpec((_BN, OUT_CH // 2), lambda i: (i, 0)),
        pl.BlockSpec((_BN, OUT_CH // 2), lambda i: (i, 0)),
    ],
    out_shape=[jax.ShapeDtypeStruct((N, OUT_CH // 2), jnp.bfloat16)] * 2,
)


def _interleave_perm(C):
    p = []
    for base in range(0, C, 2 * L):
        for j in range(L):
            p.append(base + j)
            p.append(base + L + j)
    return jnp.array(p, dtype=jnp.int32)


@jax.jit
def kernel(x, edge_index, edge_weight, W1, b1, W2, b2):
    pad = EPAD - E
    izero = jnp.zeros((pad,), jnp.int32)
    src2d = jnp.concatenate([edge_index[0], izero]).reshape(EPAD // K, K)
    dst2d = jnp.concatenate([edge_index[1], izero]).reshape(EPAD // SK, SK)
    wpad = jnp.concatenate([edge_weight, jnp.zeros((pad,), jnp.float32)])

    HH = HID_CH // 2
    OH = OUT_CH // 2
    permH = _interleave_perm(HH)
    permO = _interleave_perm(OH)
    W1s = jnp.stack([W1[:, :HH][:, permH], W1[:, HH:][:, permH]])
    full_perm = jnp.concatenate([permO, OH + permO])
    W2s = jnp.stack([W2[:HH], W2[HH:]])[:, :, full_perm]
    b1s = b1.reshape(2, 1, HH)
    init1 = jnp.zeros((NC, RPT_LAST, HH), jnp.float32)
    init2 = jnp.broadcast_to(b2.reshape(2, 1, OH), (NC, RPT_LAST, OH))

    h1a, h1b = _mm1(x, W1s)
    agg1 = _agg_hid(h1a, h1b, src2d, dst2d, wpad, init1)
    h2a, h2b = _mm2(agg1, b1s, W2s)
    agg2 = _agg_out(h2a, h2b, src2d, dst2d, wpad, init2)
    return jnp.concatenate([agg2[0], agg2[1]], axis=1)


# speedup vs baseline: 1.0113x; 1.0113x over previous
import functools

import jax
import jax.numpy as jnp
from jax import lax
from jax.experimental import pallas as pl
from jax.experimental.pallas import tpu as pltpu
from jax.experimental.pallas import tpu_sc as plsc

N = 10000
E = 320000
IN_CH = 128
HID_CH = 256
OUT_CH = 128

NC = 2
NS = 16
L = 16

K = 64
CHT = 320
EPT = CHT * K
EPAD = NS * EPT
G = 8
GK = G * K
NGRP = CHT // G
NBUF = 4
RPT_A = 624
RPT_LAST = N - (NS - 1) * RPT_A


def _mm1_body(x_ref, w_ref, o0_ref, o1_ref):
    x = x_ref[...]
    o0_ref[...] = jnp.dot(
        x, w_ref[0], preferred_element_type=jnp.float32).astype(jnp.bfloat16)
    o1_ref[...] = jnp.dot(
        x, w_ref[1], preferred_element_type=jnp.float32).astype(jnp.bfloat16)


def _mm2_body(a_ref, b1_ref, w_ref, o0_ref, o1_ref):
    a0 = jnp.maximum(a_ref[0] + b1_ref[0], 0.0)
    a1 = jnp.maximum(a_ref[1] + b1_ref[1], 0.0)
    h = (jnp.dot(a0, w_ref[0], preferred_element_type=jnp.float32)
         + jnp.dot(a1, w_ref[1], preferred_element_type=jnp.float32))
    o0_ref[...] = h[:, :OUT_CH // 2].astype(jnp.bfloat16)
    o1_ref[...] = h[:, OUT_CH // 2:].astype(jnp.bfloat16)


def _make_agg(C):
    mesh = plsc.VectorSubcoreMesh(core_axis_name="c", subcore_axis_name="s")

    @functools.partial(
        pl.kernel,
        out_type=jax.ShapeDtypeStruct((NC, N, C), jnp.float32),
        mesh=mesh,
        compiler_params=pltpu.CompilerParams(
            needs_layout_passes=False, use_tc_tiling_on_sc=False),
        scratch_types=[
            pltpu.VMEM((2, G, K), jnp.int32),
            pltpu.VMEM((2, G, K), jnp.int32),
            pltpu.VMEM((2 * GK,), jnp.float32),
            pltpu.VMEM((NBUF, K, C), jnp.bfloat16),
            pltpu.VMEM((2, K, C), jnp.float32),
            pltpu.VMEM_SHARED((N, C), jnp.float32),
            [pltpu.SemaphoreType.DMA] * NBUF,
            [pltpu.SemaphoreType.DMA] * 2,
            [pltpu.SemaphoreType.DMA] * 2,
            [pltpu.SemaphoreType.DMA] * 2,
            [pltpu.SemaphoreType.DMA] * 2,
        ],
    )
    def agg(tab0, tab1, src2d, dst2d, wflat, init, out,
            srcv, dstv, wv, rows, outb, acc, gsems, ssems, isems, jsems,
            ksems):
        cid = lax.axis_index("c")
        sid = lax.axis_index("s")
        stripe_a = pl.ds(sid * RPT_A, RPT_A)
        stripe_l = pl.ds((NS - 1) * RPT_A, RPT_LAST)

        @pl.when(sid < NS - 1)
        def _():
            pltpu.sync_copy(init.at[cid, pl.ds(0, RPT_A)], acc.at[stripe_a])

        @pl.when(sid == NS - 1)
        def _():
            pltpu.sync_copy(init.at[cid], acc.at[stripe_l])

        def istart(g, p):
            r0 = sid * CHT + g * G
            pltpu.async_copy(src2d.at[pl.ds(r0, G)], srcv.at[p], isems[p])
            pltpu.async_copy(dst2d.at[pl.ds(r0, G)], dstv.at[p], jsems[p])
            pltpu.async_copy(wflat.at[pl.ds(r0 * K, GK)],
                             wv.at[pl.ds(p * GK, GK)], ksems[p])

        def iwait(p):
            pltpu.make_async_copy(
                src2d.at[pl.ds(0, G)], srcv.at[p], isems[p]).wait()
            pltpu.make_async_copy(
                dst2d.at[pl.ds(0, G)], dstv.at[p], jsems[p]).wait()
            pltpu.make_async_copy(
                wflat.at[pl.ds(0, GK)], wv.at[pl.ds(p * GK, GK)],
                ksems[p]).wait()

        def gstart(p, brow, q):
            @pl.when(cid == 0)
            def _():
                pltpu.async_copy(
                    tab0.at[srcv.at[p, brow]], rows.at[q], gsems[q])

            @pl.when(cid == 1)
            def _():
                pltpu.async_copy(
                    tab1.at[srcv.at[p, brow]], rows.at[q], gsems[q])

        def gwait(q):
            pltpu.make_async_copy(
                tab0.at[srcv.at[0, 0]], rows.at[q], gsems[q]).wait()

        def sstart(p, brow, o):
            pltpu.async_copy(
                outb.at[o], acc.at[dstv.at[p, brow]], ssems[o], add=True)

        def swait_out(o):
            pltpu.make_async_copy(
                outb.at[o], acc.at[dstv.at[0, 0]], ssems[o]).wait()

        def scale(p, b, q, o):
            wbase = p * GK + b * K

            @pl.loop(0, K, unroll=4)
            def _edge(i):
                wspl = plsc.load_gather(
                    wv, [jnp.full((L,), wbase + i, jnp.int32)])
                for cb in range(C // (2 * L)):
                    v = rows[q, i, pl.ds(cb * 2 * L, 2 * L)]
                    va, vb = plsc.unpack(
                        v, format=plsc.PackFormat.INTERLEAVED)
                    outb[o, i, pl.ds(cb * 2 * L, L)] = va * wspl
                    outb[o, i, pl.ds(cb * 2 * L + L, L)] = vb * wspl

        istart(0, 0)
        istart(1, 1)
        iwait(0)
        gstart(0, 0, 0)
        gstart(0, 1, 1)
        gstart(0, 2, 2)
        plsc.subcore_barrier()

        @pl.loop(0, NGRP, step=2)
        def _grp(g):
            for pp in range(2):
                gg = g + pp
                for b in range(G):
                    t = gg * G + b
                    q = b % NBUF
                    o = b % 2
                    gwait(q)

                    @pl.when(t >= 2)
                    def _():
                        swait_out(o)

                    scale(pp, b, q, o)
                    sstart(pp, b, o)
                    if b == 2:
                        @pl.when((gg >= 1) & (gg + 1 < NGRP))
                        def _():
                            istart(gg + 1, 1 - pp)
                    if b == 5:
                        @pl.when(gg + 1 < NGRP)
                        def _():
                            iwait(1 - pp)
                    pn = pp if b < G - 3 else 1 - pp
                    brow = (b + 3) % G

                    @pl.when(t + 3 < CHT)
                    def _():
                        gstart(pn, brow, (b + 3) % NBUF)

        swait_out(0)
        swait_out(1)
        plsc.subcore_barrier()

        @pl.when(sid < NS - 1)
        def _():
            pltpu.sync_copy(acc.at[stripe_a], out.at[cid, stripe_a])

        @pl.when(sid == NS - 1)
        def _():
            pltpu.sync_copy(acc.at[stripe_l], out.at[cid, stripe_l])

    return agg


_agg_hid = _make_agg(HID_CH // 2)
_agg_out = _make_agg(OUT_CH // 2)

_BN = 1000

_mm1 = pl.pallas_call(
    _mm1_body,
    grid=(N // _BN,),
    in_specs=[
        pl.BlockSpec((_BN, IN_CH), lambda i: (i, 0)),
        pl.BlockSpec((2, IN_CH, HID_CH // 2), lambda i: (0, 0, 0)),
    ],
    out_specs=[
        pl.BlockSpec((_BN, HID_CH // 2), lambda i: (i, 0)),
        pl.BlockSpec((_BN, HID_CH // 2), lambda i: (i, 0)),
    ],
    out_shape=[jax.ShapeDtypeStruct((N, HID_CH // 2), jnp.bfloat16)] * 2,
)

_mm2 = pl.pallas_call(
    _mm2_body,
    grid=(N // _BN,),
    in_specs=[
        pl.BlockSpec((2, _BN, HID_CH // 2), lambda i: (0, i, 0)),
        pl.BlockSpec((2, 1, HID_CH // 2), lambda i: (0, 0, 0)),
        pl.BlockSpec((2, HID_CH // 2, OUT_CH), lambda i: (0, 0, 0)),
    ],
    out_specs=[
        pl.BlockSpec((_BN, OUT_CH // 2), lambda i: (i, 0)),
        pl.BlockSpec((_BN, OUT_CH // 2), lambda i: (i, 0)),
    ],
    out_shape=[jax.ShapeDtypeStruct((N, OUT_CH // 2), jnp.bfloat16)] * 2,
)


def _interleave_perm(C):
    p = []
    for base in range(0, C, 2 * L):
        for j in range(L):
            p.append(base + j)
            p.append(base + L + j)
    return jnp.array(p, dtype=jnp.int32)


@jax.jit
def kernel(x, edge_index, edge_weight, W1, b1, W2, b2):
    pad = EPAD - E
    izero = jnp.zeros((pad,), jnp.int32)
    src2d = jnp.concatenate([edge_index[0], izero]).reshape(EPAD // K, K)
    dst2d = jnp.concatenate([edge_index[1], izero]).reshape(EPAD // K, K)
    wpad = jnp.concatenate([edge_weight, jnp.zeros((pad,), jnp.float32)])

    HH = HID_CH // 2
    OH = OUT_CH // 2
    permH = _interleave_perm(HH)
    permO = _interleave_perm(OH)
    W1s = jnp.stack([W1[:, :HH][:, permH], W1[:, HH:][:, permH]])
    full_perm = jnp.concatenate([permO, OH + permO])
    W2s = jnp.stack([W2[:HH], W2[HH:]])[:, :, full_perm]
    b1s = b1.reshape(2, 1, HH)
    init1 = jnp.zeros((NC, RPT_LAST, HH), jnp.float32)
    init2 = jnp.broadcast_to(b2.reshape(2, 1, OH), (NC, RPT_LAST, OH))

    h1a, h1b = _mm1(x, W1s)
    agg1 = _agg_hid(h1a, h1b, src2d, dst2d, wpad, init1)
    h2a, h2b = _mm2(agg1, b1s, W2s)
    agg2 = _agg_out(h2a, h2b, src2d, dst2d, wpad, init2)
    return jnp.concatenate([agg2[0], agg2[1]], axis=1)
